# baseline (speedup 1.0000x reference)
import dataclasses

import jax
import jax.numpy as jnp
from jax import lax
from jax.experimental import pallas as pl
from jax.experimental.pallas import tpu as pltpu
from jax.experimental.pallas import tpu_sc as plsc

N = 320000
D = 128
L = 16
NW = 32
ROWS_PER_W = N // NW
SBLK = 400
SFLT = SBLK * 4
NSB = ROWS_PER_W // SBLK
NGRP = SBLK // L
B_TC = 1600


def _compiler_params():
    cp = pltpu.CompilerParams()
    if "needs_layout_passes" in pltpu.CompilerParams.__dataclass_fields__:
        cp = dataclasses.replace(cp, needs_layout_passes=False)
    return cp


def _sc_row_sumsq(score_flat):
    mesh = plsc.VectorSubcoreMesh(core_axis_name="core", subcore_axis_name="subcore")

    @pl.kernel(
        out_type=jax.ShapeDtypeStruct((N * 4,), jnp.float32),
        mesh=mesh,
        scratch_types=[
            pltpu.VMEM((SFLT,), jnp.float32),
            pltpu.VMEM((SFLT,), jnp.float32),
            pltpu.VMEM((SFLT,), jnp.float32),
            pltpu.VMEM((SFLT,), jnp.float32),
            pltpu.SemaphoreType.DMA,
            pltpu.SemaphoreType.DMA,
            pltpu.SemaphoreType.DMA,
            pltpu.SemaphoreType.DMA,
        ],
        compiler_params=_compiler_params(),
    )
    def sc_kernel(score_hbm, ssq_hbm, sb0, sb1, qb0, qb1, si0, si1, so0, so1):
        wid = lax.axis_index("subcore") * 2 + lax.axis_index("core")
        base = wid * ROWS_PER_W
        sfbase = base * 4
        iota = lax.iota(jnp.int32, L)
        iota4 = iota * 4

        sbufs = (sb0, sb1)
        qbufs = (qb0, qb1)
        sins = (si0, si1)
        souts = (so0, so1)

        def start_in(i, b):
            cp = pltpu.make_async_copy(
                score_hbm.at[pl.ds(sfbase + i * SFLT, SFLT)], sbufs[b], sins[b]
            )
            cp.start()
            return cp

        def mk_out(i, b):
            return pltpu.make_async_copy(
                qbufs[b], ssq_hbm.at[pl.ds(sfbase + i * SFLT, SFLT)], souts[b]
            )

        zero_v = jnp.zeros((L,), jnp.float32)
        for qb in (qb0, qb1):
            @pl.loop(0, SFLT // L)
            def _(k, qb=qb):
                qb[pl.ds(k * L, L)] = zero_v

        in_copies = [None, None]
        out_copies = [None, None]
        in_copies[0] = start_in(0, 0)

        for i in range(NSB):
            b = i % 2
            nb = (i + 1) % 2
            if out_copies[nb] is not None:
                out_copies[nb].wait()
                out_copies[nb] = None
            if i + 1 < NSB:
                in_copies[nb] = start_in(i + 1, nb)
            in_copies[b].wait()
            sbuf = sbufs[b]
            qbuf = qbufs[b]

            @pl.loop(0, NGRP)
            def _(g, sbuf=sbuf, qbuf=qbuf):
                gbase = jnp.full((L,), g * (L * 4), jnp.int32) + iota4
                c0 = plsc.load_gather(sbuf, [gbase])
                c1 = plsc.load_gather(sbuf, [gbase + 1])
                c2 = plsc.load_gather(sbuf, [gbase + 2])
                c3 = plsc.load_gather(sbuf, [gbase + 3])
                ssq = c0 * c0 + c1 * c1 + c2 * c2 + c3 * c3
                plsc.store_scatter(qbuf, [gbase], ssq)

            cout = mk_out(i, b)
            cout.start()
            out_copies[b] = cout

        for b in range(2):
            if out_copies[b] is not None:
                out_copies[b].wait()

    return sc_kernel(score_flat)


def _tc_masked_copy(ssq, y):
    def body(ssq4_ref, y_ref, o_ref):
        m4 = jnp.where(ssq4_ref[...] >= 4.0, 1.0, 0.0)
        ones = jnp.ones((4, D), jnp.float32)
        mb = jax.lax.dot_general(
            m4, ones, (((1,), (0,)), ((), ())),
            preferred_element_type=jnp.float32,
        )
        o_ref[...] = y_ref[...] * mb

    return pl.pallas_call(
        body,
        grid=(N // B_TC,),
        in_specs=[
            pl.BlockSpec((B_TC, 4), lambda i: (i, 0)),
            pl.BlockSpec((B_TC, D), lambda i: (i, 0)),
        ],
        out_specs=pl.BlockSpec((B_TC, D), lambda i: (i, 0)),
        out_shape=jax.ShapeDtypeStruct((N, D), jnp.float32),
    )(ssq.reshape(N, 4), y)


def kernel(edge_index, score, y):
    del edge_index
    score_flat = score.reshape(N * 4)
    ssq = _sc_row_sumsq(score_flat)
    return _tc_masked_copy(ssq, y)
